# baseline (speedup 1.0000x reference)
import functools

import jax
import jax.numpy as jnp
from jax import lax
from jax.experimental import pallas as pl
from jax.experimental.pallas import tpu as pltpu
from jax.experimental.pallas import tpu_sc as plsc

_N = 32768
_D = 768
_E = 64
_B = 4096
_G = _N // _B

_NTILE = 16
_CHUNK = _N // _NTILE
_ROWS = _CHUNK // 16


def _router_body(x_ref, wg_ref,
                 g1_ref, idx_ref, laux_ref, ent_ref,
                 cntacc_ref, meacc_ref, entacc_ref):
    i = pl.program_id(0)

    @pl.when(i == 0)
    def _():
        cntacc_ref[...] = jnp.zeros_like(cntacc_ref)
        meacc_ref[...] = jnp.zeros_like(meacc_ref)
        entacc_ref[...] = jnp.zeros_like(entacc_ref)

    lt = jax.lax.dot_general(wg_ref[...], x_ref[...],
                             (((1,), (1,)), ((), ())),
                             preferred_element_type=jnp.float32)
    m = jnp.max(lt, axis=0, keepdims=True)
    d = lt - m
    e = jnp.exp(d)
    s = jnp.sum(e, axis=0, keepdims=True)
    t1 = jnp.sum(e * d, axis=0, keepdims=True)
    r = 1.0 / s
    entacc_ref[...] += jnp.log(s) - t1 * r
    g1_ref[...] = r.reshape(1, 1, _B)
    meacc_ref[...] += e * r

    row = jax.lax.broadcasted_iota(jnp.int32, (_E, _B), 0)
    idx = jnp.min(jnp.where(lt == m, row, _E), axis=0, keepdims=True)
    idx_ref[...] = idx.reshape(1, 1, _B)
    cntacc_ref[...] += (row == idx).astype(jnp.float32)

    @pl.when(i == _G - 1)
    def _():
        me = jnp.sum(meacc_ref[...], axis=1, keepdims=True)
        cnt = jnp.sum(cntacc_ref[...], axis=1, keepdims=True)
        laux_ref[...] = (jnp.sum(me * cnt, axis=0, keepdims=True)
                         * (_E / (_N * _N)))
        ent_ref[...] = jnp.sum(entacc_ref[...], axis=1, keepdims=True) / _N


_sc_mesh = plsc.VectorSubcoreMesh(core_axis_name="c", subcore_axis_name="s")


@functools.partial(
    pl.kernel,
    mesh=_sc_mesh,
    out_type=(jax.ShapeDtypeStruct((_N // 16, 16), jnp.int32),
              jax.ShapeDtypeStruct((_NTILE, _E), jnp.int32)),
    scratch_types=[
        pltpu.VMEM((_ROWS, 16), jnp.int32),
        pltpu.VMEM((_ROWS, 16), jnp.int32),
        pltpu.VMEM((_E,), jnp.int32),
    ],
    compiler_params=pltpu.CompilerParams(needs_layout_passes=False),
)
def _ranks_sc(idx_hbm, rank_hbm, hist_hbm, idx3, rank3, counts):
    cid = lax.axis_index("c")
    sid = lax.axis_index("s")

    @pl.when(cid == 0)
    def _():
        r0 = sid * _ROWS
        pltpu.sync_copy(idx_hbm.at[pl.ds(r0, _ROWS), :], idx3)
        for k in range(_E // 16):
            counts[pl.ds(k * 16, 16)] = jnp.zeros((16,), jnp.int32)

        def step(t, carry):
            v = idx3[t, :]
            c = plsc.load_gather(counts, [v])
            dup, last = plsc.scan_count(v)
            rank3[t, :] = c + dup - 1
            plsc.store_scatter(counts, [v], c + dup, mask=last)
            return carry

        lax.fori_loop(0, _ROWS, step, 0)
        pltpu.sync_copy(rank3, rank_hbm.at[pl.ds(r0, _ROWS), :])
        pltpu.sync_copy(counts, hist_hbm.at[sid])


@functools.partial(
    pl.kernel,
    mesh=_sc_mesh,
    out_type=jax.ShapeDtypeStruct((_N // 16, 16), jnp.int32),
    scratch_types=[
        pltpu.VMEM((_ROWS, 16), jnp.int32),
        pltpu.VMEM((_ROWS, 16), jnp.int32),
        pltpu.VMEM((_NTILE, _E), jnp.int32),
        pltpu.VMEM((_E,), jnp.int32),
    ],
    compiler_params=pltpu.CompilerParams(needs_layout_passes=False),
)
def _combine_sc(idx_hbm, rank_hbm, hist_hbm, out_hbm, idx3, rank3, allh,
                basev):
    cid = lax.axis_index("c")
    sid = lax.axis_index("s")

    @pl.when(cid == 0)
    def _():
        r0 = sid * _ROWS
        pltpu.sync_copy(idx_hbm.at[pl.ds(r0, _ROWS), :], idx3)
        pltpu.sync_copy(rank_hbm.at[pl.ds(r0, _ROWS), :], rank3)
        pltpu.sync_copy(hist_hbm, allh)

        sidv = jnp.full((16,), sid, jnp.int32)
        for k in range(_E // 16):
            acc = jnp.zeros((16,), jnp.int32)
            for tp in range(_NTILE - 1):
                rowv = allh[tp, pl.ds(k * 16, 16)]
                tpv = jnp.full((16,), tp, jnp.int32)
                acc = acc + jnp.where(tpv < sidv, rowv,
                                      jnp.zeros((16,), jnp.int32))
            basev[pl.ds(k * 16, 16)] = acc

        def fix(t, carry):
            v = idx3[t, :]
            rank3[t, :] = rank3[t, :] + plsc.load_gather(basev, [v])
            return carry

        lax.fori_loop(0, _ROWS, fix, 0)
        pltpu.sync_copy(rank3, out_hbm.at[pl.ds(r0, _ROWS), :])


def _run_tc(input, wg, interpret=False):
    g1, idx, laux, ent = pl.pallas_call(
        _router_body,
        grid=(_G,),
        in_specs=[
            pl.BlockSpec((_B, _D), lambda i: (i, 0)),
            pl.BlockSpec((_E, _D), lambda i: (0, 0)),
        ],
        out_specs=[
            pl.BlockSpec((1, 1, _B), lambda i: (i, 0, 0)),
            pl.BlockSpec((1, 1, _B), lambda i: (i, 0, 0)),
            pl.BlockSpec((1, 1), lambda i: (0, 0)),
            pl.BlockSpec((1, 1), lambda i: (0, 0)),
        ],
        out_shape=[
            jax.ShapeDtypeStruct((_G, 1, _B), jnp.float32),
            jax.ShapeDtypeStruct((_G, 1, _B), jnp.int32),
            jax.ShapeDtypeStruct((1, 1), jnp.float32),
            jax.ShapeDtypeStruct((1, 1), jnp.float32),
        ],
        scratch_shapes=[
            pltpu.VMEM((_E, _B), jnp.float32),
            pltpu.VMEM((_E, _B), jnp.float32),
            pltpu.VMEM((1, _B), jnp.float32),
        ],
        compiler_params=pltpu.CompilerParams(
            dimension_semantics=("arbitrary",),
        ),
        interpret=interpret,
    )(input, wg)
    return g1, idx, laux, ent


def kernel(input, wg):
    g1, idx, laux, ent = _run_tc(input, wg)
    idx2 = idx.reshape(_N // 16, 16)
    rank, hist = _ranks_sc(idx2)
    loc = _combine_sc(idx2, rank, hist)
    return (laux.reshape(()), g1.reshape(_N), idx.reshape(_N),
            loc.reshape(_N), ent.reshape(()))


# speedup vs baseline: 1.0201x; 1.0201x over previous
import functools

import jax
import jax.numpy as jnp
from jax import lax
from jax.experimental import pallas as pl
from jax.experimental.pallas import tpu as pltpu
from jax.experimental.pallas import tpu_sc as plsc

_N = 32768
_D = 768
_E = 64
_B = 4096
_G = _N // _B

_NTILE = 32
_CHUNK = _N // _NTILE
_ROWS = _CHUNK // 16


def _router_body(x_ref, wg_ref,
                 g1_ref, idx_ref, laux_ref, ent_ref,
                 cntacc_ref, meacc_ref, entacc_ref):
    i = pl.program_id(0)

    @pl.when(i == 0)
    def _():
        cntacc_ref[...] = jnp.zeros_like(cntacc_ref)
        meacc_ref[...] = jnp.zeros_like(meacc_ref)
        entacc_ref[...] = jnp.zeros_like(entacc_ref)

    lt = jax.lax.dot_general(wg_ref[...], x_ref[...],
                             (((1,), (1,)), ((), ())),
                             preferred_element_type=jnp.float32)
    m = jnp.max(lt, axis=0, keepdims=True)
    d = lt - m
    e = jnp.exp(d)
    s = jnp.sum(e, axis=0, keepdims=True)
    t1 = jnp.sum(e * d, axis=0, keepdims=True)
    r = 1.0 / s
    entacc_ref[...] += jnp.log(s) - t1 * r
    g1_ref[...] = r.reshape(1, 1, _B)
    meacc_ref[...] += e * r

    row = jax.lax.broadcasted_iota(jnp.int32, (_E, _B), 0)
    idx = jnp.min(jnp.where(lt == m, row, _E), axis=0, keepdims=True)
    idx_ref[...] = idx.reshape(1, 1, _B)
    cntacc_ref[...] += (row == idx).astype(jnp.float32)

    @pl.when(i == _G - 1)
    def _():
        me = jnp.sum(meacc_ref[...], axis=1, keepdims=True)
        cnt = jnp.sum(cntacc_ref[...], axis=1, keepdims=True)
        laux_ref[...] = (jnp.sum(me * cnt, axis=0, keepdims=True)
                         * (_E / (_N * _N)))
        ent_ref[...] = jnp.sum(entacc_ref[...], axis=1, keepdims=True) / _N


_sc_mesh = plsc.VectorSubcoreMesh(core_axis_name="c", subcore_axis_name="s")


@functools.partial(
    pl.kernel,
    mesh=_sc_mesh,
    out_type=(jax.ShapeDtypeStruct((_N // 16, 16), jnp.int32),
              jax.ShapeDtypeStruct((_NTILE, _E), jnp.int32)),
    scratch_types=[
        pltpu.VMEM((_ROWS, 16), jnp.int32),
        pltpu.VMEM((_ROWS, 16), jnp.int32),
        pltpu.VMEM((_E,), jnp.int32),
    ],
    compiler_params=pltpu.CompilerParams(needs_layout_passes=False),
)
def _ranks_sc(idx_hbm, rank_hbm, hist_hbm, idx3, rank3, counts):
    wid = lax.axis_index("s") * 2 + lax.axis_index("c")

    if True:
        r0 = wid * _ROWS
        pltpu.sync_copy(idx_hbm.at[pl.ds(r0, _ROWS), :], idx3)
        for k in range(_E // 16):
            counts[pl.ds(k * 16, 16)] = jnp.zeros((16,), jnp.int32)

        def step(t, carry):
            v = idx3[t, :]
            c = plsc.load_gather(counts, [v])
            dup, last = plsc.scan_count(v)
            rank3[t, :] = c + dup - 1
            plsc.store_scatter(counts, [v], c + dup, mask=last)
            return carry

        lax.fori_loop(0, _ROWS, step, 0)
        pltpu.sync_copy(rank3, rank_hbm.at[pl.ds(r0, _ROWS), :])
        pltpu.sync_copy(counts, hist_hbm.at[wid])


@functools.partial(
    pl.kernel,
    mesh=_sc_mesh,
    out_type=jax.ShapeDtypeStruct((_N // 16, 16), jnp.int32),
    scratch_types=[
        pltpu.VMEM((_ROWS, 16), jnp.int32),
        pltpu.VMEM((_ROWS, 16), jnp.int32),
        pltpu.VMEM((_NTILE, _E), jnp.int32),
        pltpu.VMEM((_E,), jnp.int32),
    ],
    compiler_params=pltpu.CompilerParams(needs_layout_passes=False),
)
def _combine_sc(idx_hbm, rank_hbm, hist_hbm, out_hbm, idx3, rank3, allh,
                basev):
    wid = lax.axis_index("s") * 2 + lax.axis_index("c")

    if True:
        r0 = wid * _ROWS
        pltpu.sync_copy(idx_hbm.at[pl.ds(r0, _ROWS), :], idx3)
        pltpu.sync_copy(rank_hbm.at[pl.ds(r0, _ROWS), :], rank3)
        pltpu.sync_copy(hist_hbm, allh)

        sidv = jnp.full((16,), wid, jnp.int32)
        for k in range(_E // 16):
            acc = jnp.zeros((16,), jnp.int32)
            for tp in range(_NTILE - 1):
                rowv = allh[tp, pl.ds(k * 16, 16)]
                tpv = jnp.full((16,), tp, jnp.int32)
                acc = acc + jnp.where(tpv < sidv, rowv,
                                      jnp.zeros((16,), jnp.int32))
            basev[pl.ds(k * 16, 16)] = acc

        def fix(t, carry):
            v = idx3[t, :]
            rank3[t, :] = rank3[t, :] + plsc.load_gather(basev, [v])
            return carry

        lax.fori_loop(0, _ROWS, fix, 0)
        pltpu.sync_copy(rank3, out_hbm.at[pl.ds(r0, _ROWS), :])


def _run_tc(input, wg, interpret=False):
    g1, idx, laux, ent = pl.pallas_call(
        _router_body,
        grid=(_G,),
        in_specs=[
            pl.BlockSpec((_B, _D), lambda i: (i, 0)),
            pl.BlockSpec((_E, _D), lambda i: (0, 0)),
        ],
        out_specs=[
            pl.BlockSpec((1, 1, _B), lambda i: (i, 0, 0)),
            pl.BlockSpec((1, 1, _B), lambda i: (i, 0, 0)),
            pl.BlockSpec((1, 1), lambda i: (0, 0)),
            pl.BlockSpec((1, 1), lambda i: (0, 0)),
        ],
        out_shape=[
            jax.ShapeDtypeStruct((_G, 1, _B), jnp.float32),
            jax.ShapeDtypeStruct((_G, 1, _B), jnp.int32),
            jax.ShapeDtypeStruct((1, 1), jnp.float32),
            jax.ShapeDtypeStruct((1, 1), jnp.float32),
        ],
        scratch_shapes=[
            pltpu.VMEM((_E, _B), jnp.float32),
            pltpu.VMEM((_E, _B), jnp.float32),
            pltpu.VMEM((1, _B), jnp.float32),
        ],
        compiler_params=pltpu.CompilerParams(
            dimension_semantics=("arbitrary",),
        ),
        interpret=interpret,
    )(input, wg)
    return g1, idx, laux, ent


def kernel(input, wg):
    g1, idx, laux, ent = _run_tc(input, wg)
    idx2 = idx.reshape(_N // 16, 16)
    rank, hist = _ranks_sc(idx2)
    loc = _combine_sc(idx2, rank, hist)
    return (laux.reshape(()), g1.reshape(_N), idx.reshape(_N),
            loc.reshape(_N), ent.reshape(()))


# speedup vs baseline: 1.0210x; 1.0009x over previous
import functools

import jax
import jax.numpy as jnp
from jax import lax
from jax.experimental import pallas as pl
from jax.experimental.pallas import tpu as pltpu
from jax.experimental.pallas import tpu_sc as plsc

_N = 32768
_D = 768
_E = 64
_B = 4096
_G = _N // _B

_NTILE = 32
_CHUNK = _N // _NTILE
_ROWS = _CHUNK // 16


def _router_body(x_ref, wg_ref,
                 g1_ref, idx_ref, laux_ref, ent_ref,
                 cntacc_ref, meacc_ref, entacc_ref):
    i = pl.program_id(0)

    @pl.when(i == 0)
    def _():
        cntacc_ref[...] = jnp.zeros_like(cntacc_ref)
        meacc_ref[...] = jnp.zeros_like(meacc_ref)
        entacc_ref[...] = jnp.zeros_like(entacc_ref)

    lt = jax.lax.dot_general(wg_ref[...], x_ref[...],
                             (((1,), (1,)), ((), ())),
                             preferred_element_type=jnp.float32)
    m = jnp.max(lt, axis=0, keepdims=True)
    d = lt - m
    e = jnp.exp(d)
    s = jnp.sum(e, axis=0, keepdims=True)
    t1 = jnp.sum(e * d, axis=0, keepdims=True)
    r = 1.0 / s
    entacc_ref[...] += jnp.log(s) - t1 * r
    g1_ref[...] = r.reshape(1, 1, _B)
    meacc_ref[...] += e * r

    row = jax.lax.broadcasted_iota(jnp.int32, (_E, _B), 0)
    idx = jnp.min(jnp.where(lt == m, row, _E), axis=0, keepdims=True)
    idx_ref[...] = idx.reshape(1, 1, _B)
    cntacc_ref[...] += (row == idx).astype(jnp.float32)

    @pl.when(i == _G - 1)
    def _():
        me = jnp.sum(meacc_ref[...], axis=1, keepdims=True)
        cnt = jnp.sum(cntacc_ref[...], axis=1, keepdims=True)
        laux_ref[...] = (jnp.sum(me * cnt, axis=0, keepdims=True)
                         * (_E / (_N * _N)))
        ent_ref[...] = jnp.sum(entacc_ref[...], axis=1, keepdims=True) / _N


_sc_mesh = plsc.VectorSubcoreMesh(core_axis_name="c", subcore_axis_name="s")


@functools.partial(
    pl.kernel,
    mesh=_sc_mesh,
    out_type=(jax.ShapeDtypeStruct((_N // 16, 16), jnp.int32),
              jax.ShapeDtypeStruct((_NTILE, _E), jnp.int32)),
    scratch_types=[
        pltpu.VMEM((_ROWS, 16), jnp.int32),
        pltpu.VMEM((_ROWS, 16), jnp.int32),
        pltpu.VMEM((_E,), jnp.int32),
    ],
    compiler_params=pltpu.CompilerParams(needs_layout_passes=False),
)
def _ranks_sc(idx_hbm, rank_hbm, hist_hbm, idx3, rank3, counts):
    wid = lax.axis_index("s") * 2 + lax.axis_index("c")

    r0 = wid * _ROWS
    pltpu.sync_copy(idx_hbm.at[pl.ds(r0, _ROWS), :], idx3)
    for k in range(_E // 16):
        counts[pl.ds(k * 16, 16)] = jnp.zeros((16,), jnp.int32)

    def step(t, carry):
        v = idx3[t, :]
        c = plsc.load_gather(counts, [v])
        dup, last = plsc.scan_count(v)
        rank3[t, :] = c + dup - 1
        plsc.store_scatter(counts, [v], c + dup, mask=last)
        return carry

    lax.fori_loop(0, _ROWS, step, 0)
    pltpu.sync_copy(rank3, rank_hbm.at[pl.ds(r0, _ROWS), :])
    pltpu.sync_copy(counts, hist_hbm.at[wid])


@functools.partial(
    pl.kernel,
    mesh=_sc_mesh,
    out_type=jax.ShapeDtypeStruct((_N // 16, 16), jnp.int32),
    scratch_types=[
        pltpu.VMEM((_ROWS, 16), jnp.int32),
        pltpu.VMEM((_ROWS, 16), jnp.int32),
        pltpu.VMEM((_NTILE, _E), jnp.int32),
        pltpu.VMEM((_E,), jnp.int32),
    ],
    compiler_params=pltpu.CompilerParams(needs_layout_passes=False),
)
def _combine_sc(idx_hbm, rank_hbm, hist_hbm, out_hbm, idx3, rank3, allh,
            basev):
    wid = lax.axis_index("s") * 2 + lax.axis_index("c")

    r0 = wid * _ROWS
    pltpu.sync_copy(idx_hbm.at[pl.ds(r0, _ROWS), :], idx3)
    pltpu.sync_copy(rank_hbm.at[pl.ds(r0, _ROWS), :], rank3)
    pltpu.sync_copy(hist_hbm, allh)

    sidv = jnp.full((16,), wid, jnp.int32)
    for k in range(_E // 16):
        acc = jnp.zeros((16,), jnp.int32)
        for tp in range(_NTILE - 1):
            rowv = allh[tp, pl.ds(k * 16, 16)]
            tpv = jnp.full((16,), tp, jnp.int32)
            acc = acc + jnp.where(tpv < sidv, rowv,
                                  jnp.zeros((16,), jnp.int32))
        basev[pl.ds(k * 16, 16)] = acc

    def fix(t, carry):
        v = idx3[t, :]
        rank3[t, :] = rank3[t, :] + plsc.load_gather(basev, [v])
        return carry

    lax.fori_loop(0, _ROWS, fix, 0)
    pltpu.sync_copy(rank3, out_hbm.at[pl.ds(r0, _ROWS), :])


def _run_tc(input, wg, interpret=False):
    g1, idx, laux, ent = pl.pallas_call(
        _router_body,
        grid=(_G,),
        in_specs=[
            pl.BlockSpec((_B, _D), lambda i: (i, 0)),
            pl.BlockSpec((_E, _D), lambda i: (0, 0)),
        ],
        out_specs=[
            pl.BlockSpec((1, 1, _B), lambda i: (i, 0, 0)),
            pl.BlockSpec((1, 1, _B), lambda i: (i, 0, 0)),
            pl.BlockSpec((1, 1), lambda i: (0, 0)),
            pl.BlockSpec((1, 1), lambda i: (0, 0)),
        ],
        out_shape=[
            jax.ShapeDtypeStruct((_G, 1, _B), jnp.float32),
            jax.ShapeDtypeStruct((_G, 1, _B), jnp.int32),
            jax.ShapeDtypeStruct((1, 1), jnp.float32),
            jax.ShapeDtypeStruct((1, 1), jnp.float32),
        ],
        scratch_shapes=[
            pltpu.VMEM((_E, _B), jnp.float32),
            pltpu.VMEM((_E, _B), jnp.float32),
            pltpu.VMEM((1, _B), jnp.float32),
        ],
        compiler_params=pltpu.CompilerParams(
            dimension_semantics=("arbitrary",),
        ),
        interpret=interpret,
    )(input, wg)
    return g1, idx, laux, ent


def kernel(input, wg):
    g1, idx, laux, ent = _run_tc(input, wg)
    idx2 = idx.reshape(_N // 16, 16)
    rank, hist = _ranks_sc(idx2)
    loc = _combine_sc(idx2, rank, hist)
    return (laux.reshape(()), g1.reshape(_N), idx.reshape(_N),
            loc.reshape(_N), ent.reshape(()))
